# baseline (speedup 1.0000x reference)
import functools

import jax
import jax.numpy as jnp
from jax import lax
from jax.experimental import pallas as pl
from jax.experimental.pallas import tpu as pltpu
from jax.experimental.pallas import tpu_sc as plsc

BATCH = 16384
DIM = 32
NC = 2
NS = 16
NW = NC * NS
B_PER_W = BATCH // NW
CHUNK = 128
NCHUNK = B_PER_W // CHUNK
GENDER_ROWS = 5
LANES = 16
PAD = DIM + 1


def _user_body(uid_hbm, utab_hbm, out_hbm, uidx_v, uv, sem_in, sems_g, sem_w):
    wid = lax.axis_index("s") * NC + lax.axis_index("c")
    base = wid * B_PER_W
    pltpu.async_copy(uid_hbm.at[pl.ds(base, B_PER_W)], uidx_v, sem_in).wait()
    gathers = []
    for j in range(NCHUNK):
        rows = pl.ds(j * CHUNK, CHUNK)
        gathers.append(pltpu.async_copy(
            utab_hbm.at[uidx_v.at[rows]], uv.at[rows], sems_g.at[j]))
    writes = []
    for j in range(NCHUNK):
        gathers[j].wait()
        rows = pl.ds(j * CHUNK, CHUNK)
        writes.append(pltpu.async_copy(
            uv.at[rows], out_hbm.at[pl.ds(base + j * CHUNK, CHUNK)], sem_w))
    for w in writes:
        w.wait()


def _gender_body(gid_hbm, gtab_hbm, out_hbm, gidx_v, gv, gtab_v, gtab_p,
                 sem_in, sem_w):
    wid = lax.axis_index("s") * NC + lax.axis_index("c")
    base = wid * B_PER_W
    stages = [
        pltpu.async_copy(gid_hbm.at[pl.ds(base, B_PER_W)], gidx_v, sem_in),
        pltpu.async_copy(gtab_hbm, gtab_v, sem_in),
    ]
    for s in stages:
        s.wait()
    for g in range(GENDER_ROWS):
        for h in range(DIM // LANES):
            gtab_p[g, pl.ds(h * LANES, LANES)] = gtab_v[g, pl.ds(h * LANES, LANES)]
    lane = lax.iota(jnp.int32, LANES)

    def group(i, carry):
        pos = i * LANES + lane
        rows = plsc.load_gather(gidx_v, [pos])
        for c in range(DIM):
            col = jnp.full((LANES,), c, jnp.int32)
            vals = plsc.load_gather(gtab_p, [rows, col])
            plsc.store_scatter(gv, [pos, col], vals)
        return carry

    lax.fori_loop(0, B_PER_W // LANES, group, 0)
    pltpu.async_copy(
        gv.at[:, pl.ds(0, DIM)], out_hbm.at[pl.ds(base, B_PER_W)], sem_w
    ).wait()


_user = functools.partial(
    pl.kernel,
    out_type=jax.ShapeDtypeStruct((BATCH, DIM), jnp.float32),
    mesh=plsc.VectorSubcoreMesh(core_axis_name="c", subcore_axis_name="s"),
    compiler_params=pltpu.CompilerParams(
        use_tc_tiling_on_sc=False, needs_layout_passes=False),
    scratch_types=[
        pltpu.VMEM((B_PER_W,), jnp.int32),
        pltpu.VMEM((B_PER_W, DIM), jnp.float32),
        pltpu.SemaphoreType.DMA,
        pltpu.SemaphoreType.DMA((NCHUNK,)),
        pltpu.SemaphoreType.DMA,
    ],
)(_user_body)

_gender = functools.partial(
    pl.kernel,
    out_type=jax.ShapeDtypeStruct((BATCH, DIM), jnp.float32),
    mesh=plsc.VectorSubcoreMesh(core_axis_name="c", subcore_axis_name="s"),
    compiler_params=pltpu.CompilerParams(
        use_tc_tiling_on_sc=False, needs_layout_passes=False),
    scratch_types=[
        pltpu.VMEM((B_PER_W,), jnp.int32),
        pltpu.VMEM((B_PER_W, PAD), jnp.float32),
        pltpu.VMEM((GENDER_ROWS, DIM), jnp.float32),
        pltpu.VMEM((GENDER_ROWS, PAD), jnp.float32),
        pltpu.SemaphoreType.DMA,
        pltpu.SemaphoreType.DMA,
    ],
)(_gender_body)


def kernel(customer_id, category_by_Gender, user_table, gender_table):
    user_half = _user(customer_id, user_table)
    gender_half = _gender(category_by_Gender, gender_table)
    return jnp.concatenate([user_half, gender_half], axis=1)


# speedup vs baseline: 1.0236x; 1.0236x over previous
import functools

import jax
import jax.numpy as jnp
from jax import lax
from jax.experimental import pallas as pl
from jax.experimental.pallas import tpu as pltpu
from jax.experimental.pallas import tpu_sc as plsc

BATCH = 16384
DIM = 32
NC = 2
NS = 16
NW = NC * NS
B_PER_W = BATCH // NW
CHUNK = 128
NCHUNK = B_PER_W // CHUNK
GENDER_ROWS = 5
LANES = 16


PAD = DIM + 1


def _emb_body(uid_hbm, gid_hbm, utab_hbm, gtab_hbm, out_hbm,
              uidx_v, gidx_v, uv, gv, gtab_v, gtab_p, sem_in, sems_g, sem_w):
    wid = lax.axis_index("s") * NC + lax.axis_index("c")
    base = wid * B_PER_W
    stages = [
        pltpu.async_copy(uid_hbm.at[pl.ds(base, B_PER_W)], uidx_v, sem_in),
        pltpu.async_copy(gid_hbm.at[pl.ds(base, B_PER_W)], gidx_v, sem_in),
        pltpu.async_copy(gtab_hbm, gtab_v, sem_in),
    ]
    for s in stages:
        s.wait()
    gathers = []
    for j in range(NCHUNK):
        rows = pl.ds(j * CHUNK, CHUNK)
        gathers.append(pltpu.async_copy(
            utab_hbm.at[uidx_v.at[rows]], uv.at[rows], sems_g.at[j]))
    for g in range(GENDER_ROWS):
        for h in range(DIM // LANES):
            gtab_p[g, pl.ds(h * LANES, LANES)] = gtab_v[g, pl.ds(h * LANES, LANES)]
    lane = lax.iota(jnp.int32, LANES)

    def group(i, carry):
        pos = i * LANES + lane
        rows = plsc.load_gather(gidx_v, [pos])
        for c in range(DIM):
            col = jnp.full((LANES,), c, jnp.int32)
            vals = plsc.load_gather(gtab_p, [rows, col])
            plsc.store_scatter(gv, [pos, col], vals)
        return carry

    lax.fori_loop(0, B_PER_W // LANES, group, 0)
    writes = [pltpu.async_copy(
        gv.at[:, pl.ds(0, DIM)],
        out_hbm.at[pl.ds(base, B_PER_W), pl.ds(DIM, DIM)], sem_w)]
    for j in range(NCHUNK):
        gathers[j].wait()
        rows = pl.ds(j * CHUNK, CHUNK)
        writes.append(pltpu.async_copy(
            uv.at[rows],
            out_hbm.at[pl.ds(base + j * CHUNK, CHUNK), pl.ds(0, DIM)],
            sem_w))
    for w in writes:
        w.wait()


_emb = functools.partial(
    pl.kernel,
    out_type=jax.ShapeDtypeStruct((BATCH, 2 * DIM), jnp.float32),
    mesh=plsc.VectorSubcoreMesh(core_axis_name="c", subcore_axis_name="s"),
    compiler_params=pltpu.CompilerParams(
        use_tc_tiling_on_sc=False, needs_layout_passes=False,
        skip_device_barrier=True),
    scratch_types=[
        pltpu.VMEM((B_PER_W,), jnp.int32),
        pltpu.VMEM((B_PER_W,), jnp.int32),
        pltpu.VMEM((B_PER_W, DIM), jnp.float32),
        pltpu.VMEM((B_PER_W, PAD), jnp.float32),
        pltpu.VMEM((GENDER_ROWS, DIM), jnp.float32),
        pltpu.VMEM((GENDER_ROWS, PAD), jnp.float32),
        pltpu.SemaphoreType.DMA,
        pltpu.SemaphoreType.DMA((NCHUNK,)),
        pltpu.SemaphoreType.DMA,
    ],
)(_emb_body)


def kernel(customer_id, category_by_Gender, user_table, gender_table):
    return _emb(customer_id, category_by_Gender, user_table, gender_table)


# speedup vs baseline: 1.0287x; 1.0050x over previous
import functools

import jax
import jax.numpy as jnp
from jax import lax
from jax.experimental import pallas as pl
from jax.experimental.pallas import tpu as pltpu
from jax.experimental.pallas import tpu_sc as plsc

BATCH = 16384
DIM = 32
NC = 2
NS = 16
NW = NC * NS
B_PER_W = BATCH // NW
CHUNK = 128
NCHUNK = B_PER_W // CHUNK
GENDER_ROWS = 5
LANES = 16


PAD = DIM + 1


def _emb_body(uid_hbm, gid_hbm, utab_hbm, gtab_hbm, out_hbm,
              uidx_v, gidx_v, uv, gv, gtab_v, gtab_p, sem_in, sems_g, sem_w):
    wid = lax.axis_index("s") * NC + lax.axis_index("c")
    base = wid * B_PER_W
    stages = [
        pltpu.async_copy(uid_hbm.at[pl.ds(base, B_PER_W)], uidx_v, sem_in),
        pltpu.async_copy(gid_hbm.at[pl.ds(base, B_PER_W)], gidx_v, sem_in),
        pltpu.async_copy(gtab_hbm, gtab_v, sem_in),
    ]
    for s in stages:
        s.wait()
    gathers = []
    for j in range(NCHUNK):
        rows = pl.ds(j * CHUNK, CHUNK)
        gathers.append(pltpu.async_copy(
            utab_hbm.at[uidx_v.at[rows]], uv.at[rows], sems_g.at[j]))
    for g in range(GENDER_ROWS):
        for h in range(DIM // LANES):
            gtab_p[g, pl.ds(h * LANES, LANES)] = gtab_v[g, pl.ds(h * LANES, LANES)]
    lane = lax.iota(jnp.int32, LANES)

    def group(i, carry):
        pos = i * LANES + lane
        rows = plsc.load_gather(gidx_v, [pos])
        for c in range(DIM):
            col = jnp.full((LANES,), c, jnp.int32)
            vals = plsc.load_gather(gtab_p, [rows, col])
            plsc.store_scatter(gv, [pos, col], vals)
        return carry

    lax.fori_loop(0, B_PER_W // LANES, group, 0)
    writes = [pltpu.async_copy(
        gv.at[:, pl.ds(0, DIM)],
        out_hbm.at[pl.ds(base, B_PER_W), pl.ds(DIM, DIM)], sem_w)]
    for j in range(NCHUNK):
        gathers[j].wait()
        rows = pl.ds(j * CHUNK, CHUNK)
        writes.append(pltpu.async_copy(
            uv.at[rows, pl.ds(0, DIM)],
            out_hbm.at[pl.ds(base + j * CHUNK, CHUNK), pl.ds(0, DIM)],
            sem_w))
    for w in writes:
        w.wait()


_emb = functools.partial(
    pl.kernel,
    out_type=jax.ShapeDtypeStruct((BATCH, 2 * DIM), jnp.float32),
    mesh=plsc.VectorSubcoreMesh(core_axis_name="c", subcore_axis_name="s"),
    compiler_params=pltpu.CompilerParams(
        use_tc_tiling_on_sc=False, needs_layout_passes=False,
        skip_device_barrier=True),
    scratch_types=[
        pltpu.VMEM((B_PER_W,), jnp.int32),
        pltpu.VMEM((B_PER_W,), jnp.int32),
        pltpu.VMEM((B_PER_W, 4 * DIM), jnp.float32),
        pltpu.VMEM((B_PER_W, PAD), jnp.float32),
        pltpu.VMEM((GENDER_ROWS, DIM), jnp.float32),
        pltpu.VMEM((GENDER_ROWS, PAD), jnp.float32),
        pltpu.SemaphoreType.DMA,
        pltpu.SemaphoreType.DMA((NCHUNK,)),
        pltpu.SemaphoreType.DMA,
    ],
)(_emb_body)


def kernel(customer_id, category_by_Gender, user_table, gender_table):
    utab = jnp.pad(user_table, ((0, 0), (0, 128 - DIM)))
    return _emb(customer_id, category_by_Gender, utab, gender_table)


# speedup vs baseline: 1.0317x; 1.0029x over previous
import functools

import jax
import jax.numpy as jnp
from jax import lax
from jax.experimental import pallas as pl
from jax.experimental.pallas import tpu as pltpu
from jax.experimental.pallas import tpu_sc as plsc

BATCH = 16384
DIM = 32
NC = 2
NS = 16
NW = NC * NS
B_PER_W = BATCH // NW
CHUNK = 128
NCHUNK = B_PER_W // CHUNK
GENDER_ROWS = 5
LANES = 16


PAD = DIM + 1


def _emb_body(uid_hbm, gid_hbm, utab_hbm, gtab_hbm, out_hbm,
              uidx_v, gidx_v, uv, gv, gtab_v, gtab_p, sem_in, sems_g, sem_w):
    wid = lax.axis_index("s") * NC + lax.axis_index("c")
    base = wid * B_PER_W
    stages = [
        pltpu.async_copy(uid_hbm.at[pl.ds(base, B_PER_W)], uidx_v, sem_in),
        pltpu.async_copy(gid_hbm.at[pl.ds(base, B_PER_W)], gidx_v, sem_in),
        pltpu.async_copy(gtab_hbm, gtab_v, sem_in),
    ]
    for s in stages:
        s.wait()
    gathers = []
    for j in range(NCHUNK):
        rows = pl.ds(j * CHUNK, CHUNK)
        gathers.append(pltpu.async_copy(
            utab_hbm.at[uidx_v.at[rows]], uv.at[rows], sems_g.at[j]))
    for g in range(GENDER_ROWS):
        for h in range(DIM // LANES):
            gtab_p[g, pl.ds(h * LANES, LANES)] = gtab_v[g, pl.ds(h * LANES, LANES)]
    lane = lax.iota(jnp.int32, LANES)

    def group(i, carry):
        pos = i * LANES + lane
        rows = plsc.load_gather(gidx_v, [pos])
        for c in range(DIM):
            col = jnp.full((LANES,), c, jnp.int32)
            vals = plsc.load_gather(gtab_p, [rows, col])
            plsc.store_scatter(gv, [pos, col], vals)
        return carry

    lax.fori_loop(0, B_PER_W // LANES, group, 0)
    writes = [pltpu.async_copy(
        gv.at[:, pl.ds(0, DIM)],
        out_hbm.at[pl.ds(base, B_PER_W), pl.ds(DIM, DIM)], sem_w)]
    for j in range(NCHUNK):
        gathers[j].wait()
        rows = pl.ds(j * CHUNK, CHUNK)
        writes.append(pltpu.async_copy(
            uv.at[rows, pl.ds(0, DIM)],
            out_hbm.at[pl.ds(base + j * CHUNK, CHUNK), pl.ds(0, DIM)],
            sem_w))
    for w in writes:
        w.wait()


_emb = functools.partial(
    pl.kernel,
    out_type=jax.ShapeDtypeStruct((BATCH, 2 * DIM), jnp.float32),
    mesh=plsc.VectorSubcoreMesh(core_axis_name="c", subcore_axis_name="s"),
    compiler_params=pltpu.CompilerParams(
        use_tc_tiling_on_sc=False, needs_layout_passes=False),
    scratch_types=[
        pltpu.VMEM((B_PER_W,), jnp.int32),
        pltpu.VMEM((B_PER_W,), jnp.int32),
        pltpu.VMEM((B_PER_W, 4 * DIM), jnp.float32),
        pltpu.VMEM((B_PER_W, PAD), jnp.float32),
        pltpu.VMEM((GENDER_ROWS, DIM), jnp.float32),
        pltpu.VMEM((GENDER_ROWS, PAD), jnp.float32),
        pltpu.SemaphoreType.DMA,
        pltpu.SemaphoreType.DMA((NCHUNK,)),
        pltpu.SemaphoreType.DMA,
    ],
)(_emb_body)


def kernel(customer_id, category_by_Gender, user_table, gender_table):
    utab = jnp.pad(user_table, ((0, 0), (0, 128 - DIM)))
    return _emb(customer_id, category_by_Gender, utab, gender_table)
